# baseline (speedup 1.0000x reference)
import functools

import jax
import jax.numpy as jnp
from jax import lax
from jax.experimental import pallas as pl
from jax.experimental.pallas import tpu as pltpu
from jax.experimental.pallas import tpu_sc as plsc

_NC = 2
_NS = 16
_NW = _NC * _NS
_L = 16


def _make_sc_expand(S, D):
    rows = 2 * S - 1
    slices_per_w = (2 * S) // _NW
    win = S * D
    mesh = plsc.VectorSubcoreMesh(core_axis_name="c", subcore_axis_name="s")

    @functools.partial(
        pl.kernel,
        mesh=mesh,
        out_type=jax.ShapeDtypeStruct((2 * S * S * D,), jnp.float32),
        scratch_types=[
            pltpu.VMEM((2 * S * D,), jnp.float32),
            pltpu.SemaphoreType.DMA,
        ],
    )
    def expand(table_hbm, out_hbm, buf, sem):
        cid = lax.axis_index("c")
        sid = lax.axis_index("s")
        wid = sid * _NC + cid

        pltpu.sync_copy(table_hbm, buf.at[pl.ds(0, rows * D)])

        def swap_rows(k, _):
            lo = k * D
            hi = ((rows - 1) - k) * D
            for q in range(D // _L):
                a = buf[pl.ds(lo + q * _L, _L)]
                b = buf[pl.ds(hi + q * _L, _L)]
                buf[pl.ds(lo + q * _L, _L)] = b
                buf[pl.ds(hi + q * _L, _L)] = a
            return 0

        lax.fori_loop(0, (rows - 1) // 2, swap_rows, 0)

        base = wid * slices_per_w
        i0 = lax.rem(base, S)
        copies = []
        for t in range(slices_per_w):
            off = ((S - 1) - (i0 + t)) * D
            copies.append(
                pltpu.async_copy(
                    buf.at[pl.ds(off, win)],
                    out_hbm.at[pl.ds((base + t) * win, win)],
                    sem,
                )
            )
        for cp in copies:
            cp.wait()

    return expand


def kernel(rel_pos_embedding, batch_size, seq_len):
    n_rows, D = rel_pos_embedding.shape
    S = (n_rows + 1) // 2
    static_batch = 2

    shift = (seq_len - S) + (batch_size - static_batch)
    r = jnp.arange(n_rows, dtype=jnp.int32)
    table_adj = rel_pos_embedding[jnp.clip(r + shift, 0, n_rows - 1)]

    out = _make_sc_expand(S, D)(table_adj.reshape(-1))
    return out.reshape(static_batch, S, S, D)


# speedup vs baseline: 1.0962x; 1.0962x over previous
import functools

import jax
import jax.numpy as jnp
from jax import lax
from jax.experimental import pallas as pl
from jax.experimental.pallas import tpu as pltpu
from jax.experimental.pallas import tpu_sc as plsc

_NC = 2
_NS = 16
_L = 16


def _make_sc_expand(S, D):
    rows_per_tile = (2 * S) // _NS
    slices_per_w = S // _NS
    mesh = plsc.VectorSubcoreMesh(core_axis_name="c", subcore_axis_name="s")

    @functools.partial(
        pl.kernel,
        mesh=mesh,
        out_type=jax.ShapeDtypeStruct((2, S, S, D), jnp.float32),
        scratch_types=[
            pltpu.VMEM((rows_per_tile, D), jnp.float32),
            pltpu.VMEM((rows_per_tile, D), jnp.float32),
            pltpu.VMEM_SHARED((2 * S, D), jnp.float32),
            pltpu.SemaphoreType.DMA,
        ],
    )
    def expand(table_hbm, out_hbm, tbuf, fbuf, shared, sem):
        cid = lax.axis_index("c")
        sid = lax.axis_index("s")

        src0 = (2 * S - rows_per_tile) - rows_per_tile * sid
        pltpu.sync_copy(table_hbm.at[pl.ds(src0, rows_per_tile)], tbuf)

        def flip_row(r, _):
            src = (rows_per_tile - 1) - r
            for q in range(D // _L):
                fbuf[r, pl.ds(q * _L, _L)] = tbuf[src, pl.ds(q * _L, _L)]
            return 0

        lax.fori_loop(0, rows_per_tile, flip_row, 0)
        pltpu.sync_copy(fbuf, shared.at[pl.ds(rows_per_tile * sid, rows_per_tile)])
        plsc.subcore_barrier()

        i0 = slices_per_w * sid
        copies = []
        for t in range(slices_per_w):
            off = S - (i0 + t)
            copies.append(
                pltpu.async_copy(
                    shared.at[pl.ds(off, S)],
                    out_hbm.at[cid, i0 + t],
                    sem,
                )
            )
        for cp in copies:
            cp.wait()

    return expand


def kernel(rel_pos_embedding, batch_size, seq_len):
    n_rows, D = rel_pos_embedding.shape
    S = (n_rows + 1) // 2
    static_batch = 2

    shift = (seq_len - S) + (batch_size - static_batch)
    r = jnp.arange(2 * S, dtype=jnp.int32)
    table_adj = rel_pos_embedding[jnp.clip(r + shift, 0, n_rows - 1)]

    return _make_sc_expand(S, D)(table_adj)
